# Initial kernel scaffold; baseline (speedup 1.0000x reference)
import jax
import jax.numpy as jnp
from jax.experimental import pallas as pl


def kernel(occ_so, E, b):
    raise NotImplementedError("write your pallas kernel here")


# speedup vs baseline: 15.9423x; 15.9423x over previous
import functools

import jax
import jax.numpy as jnp
from jax import lax
from jax.experimental import pallas as pl
from jax.experimental.pallas import tpu as pltpu
from jax.experimental.pallas import tpu_sc as plsc

N_SO = 100000
DIM = 64
BATCH = 16384
HIST = 50

NC = 2
NS = 16
NW = NC * NS
LANES = 16
NVR = DIM // LANES

ROWS_PER_W = BATCH // NW
BB = 16
NBLK = ROWS_PER_W // BB
IDX_PER_BLK = BB * HIST


@functools.partial(
    pl.kernel,
    out_type=jax.ShapeDtypeStruct((BATCH, DIM), jnp.float32),
    mesh=plsc.VectorSubcoreMesh(core_axis_name="c", subcore_axis_name="s"),
    scratch_types=[
        pltpu.VMEM((IDX_PER_BLK,), jnp.int32),
        pltpu.VMEM((IDX_PER_BLK, DIM), jnp.float32),
        pltpu.VMEM((DIM,), jnp.float32),
        pltpu.VMEM((BB, DIM), jnp.float32),
        pltpu.SemaphoreType.DMA,
    ],
    compiler_params=pltpu.CompilerParams(use_tc_tiling_on_sc=False),
)
def _sc_pool(occ_hbm, e_hbm, b_hbm, out_hbm, idx_v, rows_v, bias_v, out_v, sem):
    wid = lax.axis_index("s") * NC + lax.axis_index("c")
    base_row = wid * ROWS_PER_W

    pltpu.sync_copy(b_hbm, bias_v)
    bias = tuple(bias_v[pl.ds(d * LANES, LANES)] for d in range(NVR))

    def blk_body(g, carry):
        row0 = pl.multiple_of(base_row + g * BB, BB)
        idx0 = pl.multiple_of(row0 * HIST, IDX_PER_BLK)
        pltpu.sync_copy(occ_hbm.at[pl.ds(idx0, IDX_PER_BLK)], idx_v)
        pltpu.async_copy(e_hbm.at[idx_v], rows_v, sem).wait()

        def row_body(r, carry2):
            rbase = r * HIST

            def h_body(h, accs):
                row = rbase + h
                return tuple(
                    accs[d] + rows_v[row, pl.ds(d * LANES, LANES)]
                    for d in range(NVR)
                )

            accs = lax.fori_loop(0, HIST, h_body, bias, unroll=5)
            for d in range(NVR):
                out_v[r, pl.ds(d * LANES, LANES)] = accs[d]
            return carry2

        lax.fori_loop(0, BB, row_body, 0)
        pltpu.sync_copy(out_v, out_hbm.at[pl.ds(row0, BB)])
        return carry

    lax.fori_loop(0, NBLK, blk_body, 0)


def kernel(occ_so, E, b):
    occ_flat = occ_so.reshape(-1)
    return _sc_pool(occ_flat, E, b)


# speedup vs baseline: 21.9595x; 1.3774x over previous
import functools

import jax
import jax.numpy as jnp
from jax import lax
from jax.experimental import pallas as pl
from jax.experimental.pallas import tpu as pltpu
from jax.experimental.pallas import tpu_sc as plsc

N_SO = 100000
DIM = 64
BATCH = 16384
HIST = 50

NC = 2
NS = 16
NW = NC * NS
LANES = 16
NVR = DIM // LANES

ROWS_PER_W = BATCH // NW
IDX_PER_W = ROWS_PER_W * HIST
BB = 8
NBLK = ROWS_PER_W // BB
IDX_PER_BLK = BB * HIST


@functools.partial(
    pl.kernel,
    out_type=jax.ShapeDtypeStruct((BATCH, DIM), jnp.float32),
    mesh=plsc.VectorSubcoreMesh(core_axis_name="c", subcore_axis_name="s"),
    scratch_types=[
        pltpu.VMEM((IDX_PER_W,), jnp.int32),
        pltpu.VMEM((IDX_PER_BLK, DIM), jnp.float32),
        pltpu.VMEM((IDX_PER_BLK, DIM), jnp.float32),
        pltpu.VMEM((DIM,), jnp.float32),
        pltpu.VMEM((ROWS_PER_W, DIM), jnp.float32),
        pltpu.SemaphoreType.DMA,
        pltpu.SemaphoreType.DMA,
    ],
    compiler_params=pltpu.CompilerParams(use_tc_tiling_on_sc=False),
)
def _sc_pool(occ_hbm, e_hbm, b_hbm, out_hbm,
             idx_all, rows_a, rows_b, bias_v, out_all, sem_a, sem_b):
    wid = lax.axis_index("s") * NC + lax.axis_index("c")
    base_row = wid * ROWS_PER_W

    pltpu.sync_copy(
        occ_hbm.at[pl.ds(pl.multiple_of(base_row * HIST, IDX_PER_W), IDX_PER_W)],
        idx_all,
    )
    pltpu.sync_copy(b_hbm, bias_v)
    bias = tuple(bias_v[pl.ds(d * LANES, LANES)] for d in range(NVR))

    def start(g, rows_v, sem):
        idx_slice = idx_all.at[pl.ds(g * IDX_PER_BLK, IDX_PER_BLK)]
        pltpu.async_copy(e_hbm.at[idx_slice], rows_v, sem)

    def compute(g, rows_v, sem):
        pltpu.make_async_copy(
            e_hbm.at[idx_all.at[pl.ds(g * IDX_PER_BLK, IDX_PER_BLK)]],
            rows_v, sem,
        ).wait()
        row0 = g * BB

        def row_body(r, carry2):
            rbase = r * HIST

            def h_body(h, accs):
                row = rbase + h
                return tuple(
                    accs[d] + rows_v[row, pl.ds(d * LANES, LANES)]
                    for d in range(NVR)
                )

            accs = lax.fori_loop(0, HIST, h_body, bias, unroll=10)
            for d in range(NVR):
                out_all[row0 + r, pl.ds(d * LANES, LANES)] = accs[d]
            return carry2

        lax.fori_loop(0, BB, row_body, 0)

    start(0, rows_a, sem_a)

    def blk_pair(t, carry):
        g0 = t * 2
        g1 = g0 + 1
        start(g1, rows_b, sem_b)
        compute(g0, rows_a, sem_a)

        @pl.when(g1 + 1 < NBLK)
        def _():
            start(g1 + 1, rows_a, sem_a)

        compute(g1, rows_b, sem_b)
        return carry

    lax.fori_loop(0, NBLK // 2, blk_pair, 0)

    pltpu.sync_copy(
        out_all,
        out_hbm.at[pl.ds(pl.multiple_of(base_row, ROWS_PER_W), ROWS_PER_W)],
    )


def kernel(occ_so, E, b):
    occ_flat = occ_so.reshape(-1)
    return _sc_pool(occ_flat, E, b)
